# Initial kernel scaffold; baseline (speedup 1.0000x reference)
import jax
import jax.numpy as jnp
from jax.experimental import pallas as pl


def kernel(x, img_dim):
    raise NotImplementedError("write your pallas kernel here")


# speedup vs baseline: 3.1788x; 3.1788x over previous
import functools

import jax
import jax.numpy as jnp
from jax.experimental import pallas as pl
from jax.experimental.pallas import tpu as pltpu

_NA = 3
_NC = 80
_C = _NC + 5
_AW = (10.0, 16.0, 33.0)
_AH = (13.0, 30.0, 23.0)


def _yolo_body(stride_ref, x_ref, o_ref, *, g):
    i = pl.program_id(0)
    a = jax.lax.rem(i, _NA)
    v = x_ref[0]
    s = jax.nn.sigmoid(v)
    e = jnp.exp(v)
    rid = jax.lax.broadcasted_iota(jnp.int32, v.shape, 0)
    cid = jax.lax.broadcasted_iota(jnp.int32, v.shape, 1)
    stride = stride_ref[0, 0]
    gx = jax.lax.rem(cid, g).astype(jnp.float32)
    gy = jax.lax.div(cid, g).astype(jnp.float32)
    grid_off = jnp.where(rid == 0, gx, gy)
    aw = jnp.where(a == 0, _AW[0], jnp.where(a == 1, _AW[1], _AW[2]))
    ah = jnp.where(a == 0, _AH[0], jnp.where(a == 1, _AH[1], _AH[2]))
    anch = jnp.where(rid == 2, aw, ah)
    box01 = (s + grid_off) * stride
    box23 = e * anch
    out = jnp.where(rid < 2, box01, jnp.where(rid < 4, box23, s))
    o_ref[0] = out.T


def kernel(x, img_dim):
    B = x.shape[0]
    g = x.shape[2]
    hw = g * g
    n = B * _NA
    stride = (jnp.asarray(img_dim, jnp.float32) / g).reshape(1, 1)
    xv = x.reshape(n, _C, hw)
    out = pl.pallas_call(
        functools.partial(_yolo_body, g=g),
        grid=(n,),
        in_specs=[
            pl.BlockSpec(memory_space=pltpu.SMEM),
            pl.BlockSpec((1, _C, hw), lambda i: (i, 0, 0)),
        ],
        out_specs=pl.BlockSpec((1, hw, _C), lambda i: (i, 0, 0)),
        out_shape=jax.ShapeDtypeStruct((n, hw, _C), jnp.float32),
    )(stride, xv)
    return out.reshape(B, _NA * hw, _C)


# speedup vs baseline: 3.2533x; 1.0234x over previous
import functools

import jax
import jax.numpy as jnp
from jax.experimental import pallas as pl
from jax.experimental.pallas import tpu as pltpu

_NA = 3
_NC = 80
_C = _NC + 5
_CB = 5
_AW = (10.0, 16.0, 33.0)
_AH = (13.0, 30.0, 23.0)


def _yolo_body(stride_ref, x0_ref, x1_ref, x2_ref, o_ref, *, g):
    c0 = pl.program_id(0) * _CB
    stride = stride_ref[0, 0]
    hw = g * g
    for a, x_ref in enumerate((x0_ref, x1_ref, x2_ref)):
        v = x_ref[...]
        s = jax.nn.sigmoid(v)
        rid = c0 + jax.lax.broadcasted_iota(jnp.int32, v.shape, 0)
        lane = jax.lax.broadcasted_iota(jnp.int32, v.shape, 2)
        gx = jax.lax.rem(lane, g).astype(jnp.float32)
        gy = jax.lax.div(lane, g).astype(jnp.float32)
        grid_off = jnp.where(rid == 0, gx, gy)
        anch = jnp.where(rid == 2, _AW[a], _AH[a])
        box01 = (s + grid_off) * stride
        box23 = jnp.exp(v) * anch
        w = jnp.where(rid < 2, box01, jnp.where(rid < 4, box23, s))
        o_ref[:, :, a * hw:(a + 1) * hw] = w


def kernel(x, img_dim):
    B = x.shape[0]
    g = x.shape[2]
    hw = g * g
    stride = (jnp.asarray(img_dim, jnp.float32) / g).reshape(1, 1)
    y = x.reshape(B, _NA * _C, hw).transpose(1, 0, 2)
    nblk = _C // _CB
    bblk = B // 2
    out = pl.pallas_call(
        functools.partial(_yolo_body, g=g),
        grid=(nblk, B // bblk),
        in_specs=[
            pl.BlockSpec(memory_space=pltpu.SMEM),
            pl.BlockSpec((_CB, bblk, hw), lambda i, j: (i, j, 0)),
            pl.BlockSpec((_CB, bblk, hw), lambda i, j: (nblk + i, j, 0)),
            pl.BlockSpec((_CB, bblk, hw), lambda i, j: (2 * nblk + i, j, 0)),
        ],
        out_specs=pl.BlockSpec((_CB, bblk, _NA * hw), lambda i, j: (i, j, 0)),
        out_shape=jax.ShapeDtypeStruct((_C, B, _NA * hw), jnp.float32),
    )(stride, y, y, y)
    return jnp.transpose(out, (1, 2, 0))
